# Initial kernel scaffold; baseline (speedup 1.0000x reference)
import jax
import jax.numpy as jnp
from jax.experimental import pallas as pl


def kernel(query, keys):
    raise NotImplementedError("write your pallas kernel here")


# speedup vs baseline: 7.0761x; 7.0761x over previous
import functools

import jax
import jax.numpy as jnp
from jax.experimental import pallas as pl
from jax.experimental.pallas import tpu as pltpu

N_TOK = 16 * 2048
D = 128
M = 1000
MP = 1024
TILE = 2048
T = N_TOK // TILE


def _normalize_rows(x):
    n = jnp.sqrt(jnp.sum(x * x, axis=1, keepdims=True))
    return x / jnp.maximum(n, 1e-12)


def _body(q_ref, k_ref, out_ref, rowmax_ref, gi_ref, colmax_ref, acc_ref):
    p = pl.program_id(0)
    t = pl.program_id(1)

    qn = _normalize_rows(q_ref[...])

    @pl.when(p == 0)
    def _phase0():
        @pl.when(t == 0)
        def _init():
            colmax_ref[...] = jnp.full((1, MP), -jnp.inf, jnp.float32)

        s = jnp.dot(qn, k_ref[...].T, preferred_element_type=jnp.float32)
        col_ids = jax.lax.broadcasted_iota(jnp.int32, (TILE, MP), 1)
        s = jnp.where(col_ids < M, s, -jnp.inf)
        rowmax = jnp.max(s, axis=1, keepdims=True)
        gi = jnp.min(jnp.where(s == rowmax, col_ids, MP), axis=1, keepdims=True)
        lane = jax.lax.broadcasted_iota(jnp.int32, (TILE, T), 1)
        rowmax_ref[...] = jnp.where(lane == t, rowmax, rowmax_ref[...])
        gi_ref[...] = jnp.where(lane == t, gi, gi_ref[...])
        colmax_ref[...] = jnp.maximum(colmax_ref[...], jnp.max(s, axis=0, keepdims=True))

    @pl.when(p == 1)
    def _phase1():
        @pl.when(t == 0)
        def _init():
            acc_ref[...] = jnp.zeros((MP, D), jnp.float32)

        lane = jax.lax.broadcasted_iota(jnp.int32, (TILE, T), 1)
        sel = lane == t
        rowmax = jnp.sum(jnp.where(sel, rowmax_ref[...], 0.0), axis=1, keepdims=True)
        gi = jnp.sum(jnp.where(sel, gi_ref[...], 0), axis=1, keepdims=True)
        colmax = colmax_ref[...]
        col_ids = jax.lax.broadcasted_iota(jnp.int32, (TILE, MP), 1)
        onehot = col_ids == gi
        w = jnp.exp(jnp.where(onehot, rowmax - colmax, -120.0))
        w = jnp.where(onehot, w, 0.0)
        acc_ref[...] += jax.lax.dot_general(
            w, qn, (((0,), (0,)), ((), ())), preferred_element_type=jnp.float32
        )

        @pl.when(t == T - 1)
        def _finish():
            upd = 1e-05 * acc_ref[...] + k_ref[...]
            out_ref[...] = _normalize_rows(upd)


@jax.jit
def kernel(query, keys):
    q2 = query.reshape(N_TOK, D)
    kp = jnp.pad(keys, ((0, MP - M), (0, 0)))
    out = pl.pallas_call(
        _body,
        grid=(2, T),
        in_specs=[
            pl.BlockSpec((TILE, D), lambda p, t: (t, 0)),
            pl.BlockSpec((MP, D), lambda p, t: (0, 0)),
        ],
        out_specs=pl.BlockSpec((MP, D), lambda p, t: (0, 0)),
        out_shape=jax.ShapeDtypeStruct((MP, D), jnp.float32),
        scratch_shapes=[
            pltpu.VMEM((TILE, T), jnp.float32),
            pltpu.VMEM((TILE, T), jnp.int32),
            pltpu.VMEM((1, MP), jnp.float32),
            pltpu.VMEM((MP, D), jnp.float32),
        ],
    )(q2, kp)
    return out[:M]


# speedup vs baseline: 8.3558x; 1.1809x over previous
import jax
import jax.numpy as jnp
from jax.experimental import pallas as pl
from jax.experimental.pallas import tpu as pltpu

N_TOK = 16 * 2048
D = 128
M = 1000
MP = 1024
TILE = 2048
T = N_TOK // TILE


def _normalize_rows(x):
    n = jnp.sqrt(jnp.sum(x * x, axis=1, keepdims=True))
    return x / jnp.maximum(n, 1e-12)


def _body(q_ref, k_ref, kb_ref, out_ref, qs_ref, rowmax_ref, gi_ref, colmax_ref, acc_ref):
    p = pl.program_id(0)
    t = pl.program_id(1)

    @pl.when(p == 0)
    def _phase0():
        @pl.when(t == 0)
        def _init():
            colmax_ref[...] = jnp.full((1, MP), -1e30, jnp.float32)

        qb = _normalize_rows(q_ref[...]).astype(jnp.bfloat16)
        qs_ref[pl.ds(t * TILE, TILE), :] = qb
        s = jnp.dot(qb, kb_ref[...].T, preferred_element_type=jnp.float32)
        col_ids = jax.lax.broadcasted_iota(jnp.int32, (TILE, MP), 1)
        s = jnp.where(col_ids < M, s, -1e30)
        rowmax = jnp.max(s, axis=1, keepdims=True)
        gi = jnp.min(jnp.where(s == rowmax, col_ids, MP), axis=1, keepdims=True)
        lane = jax.lax.broadcasted_iota(jnp.int32, (TILE, T), 1)
        rowmax_ref[...] = jnp.where(lane == t, rowmax, rowmax_ref[...])
        gi_ref[...] = jnp.where(lane == t, gi, gi_ref[...])
        colmax_ref[...] = jnp.maximum(colmax_ref[...], jnp.max(s, axis=0, keepdims=True))

    @pl.when(p == 1)
    def _phase1():
        @pl.when(t == 0)
        def _init():
            acc_ref[...] = jnp.zeros((MP, D), jnp.float32)

        lane = jax.lax.broadcasted_iota(jnp.int32, (TILE, T), 1)
        sel = lane == t
        rowmax = jnp.sum(jnp.where(sel, rowmax_ref[...], 0.0), axis=1, keepdims=True)
        gi = jnp.sum(jnp.where(sel, gi_ref[...], 0), axis=1, keepdims=True)
        col_ids = jax.lax.broadcasted_iota(jnp.int32, (TILE, MP), 1)
        hot = col_ids == gi
        onehot = hot.astype(jnp.bfloat16)
        cm_g = jnp.max(jnp.where(hot, colmax_ref[...], -1e30), axis=1, keepdims=True)
        w = jnp.exp(rowmax - cm_g)
        qb = qs_ref[pl.ds(t * TILE, TILE), :]
        qw = (qb.astype(jnp.float32) * w).astype(jnp.bfloat16)
        acc_ref[...] += jax.lax.dot_general(
            onehot, qw, (((0,), (0,)), ((), ())), preferred_element_type=jnp.float32
        )

        @pl.when(t == T - 1)
        def _finish():
            upd = 1e-05 * acc_ref[...] + k_ref[...]
            out_ref[...] = _normalize_rows(upd)


@jax.jit
def kernel(query, keys):
    q2 = query.reshape(N_TOK, D)
    kp = jnp.pad(keys, ((0, MP - M), (0, 0)))
    out = pl.pallas_call(
        _body,
        grid=(2, T),
        in_specs=[
            pl.BlockSpec((TILE, D), lambda p, t: (jnp.where(p == 0, t, 0), 0)),
            pl.BlockSpec((MP, D), lambda p, t: (0, 0)),
            pl.BlockSpec((MP, D), lambda p, t: (0, 0)),
        ],
        out_specs=pl.BlockSpec((MP, D), lambda p, t: (0, 0)),
        out_shape=jax.ShapeDtypeStruct((MP, D), jnp.float32),
        scratch_shapes=[
            pltpu.VMEM((N_TOK, D), jnp.bfloat16),
            pltpu.VMEM((TILE, T), jnp.float32),
            pltpu.VMEM((TILE, T), jnp.int32),
            pltpu.VMEM((1, MP), jnp.float32),
            pltpu.VMEM((MP, D), jnp.float32),
        ],
    )(q2, kp, kp.astype(jnp.bfloat16))
    return out[:M]


# speedup vs baseline: 12.2872x; 1.4705x over previous
import jax
import jax.numpy as jnp
from jax.experimental import pallas as pl
from jax.experimental.pallas import tpu as pltpu

N_TOK = 16 * 2048
D = 128
M = 1000
MP = 1024
TILE = 2048
T = N_TOK // TILE


def _body(q_ref, kb_ref, kt_ref, out_ref, colmax_ref, acc_ref):
    t = pl.program_id(0)

    @pl.when(t == 0)
    def _init():
        colmax_ref[...] = jnp.full((1, MP), -1e30, jnp.float32)
        acc_ref[...] = jnp.zeros((D, MP), jnp.float32)

    q = q_ref[...]
    qn = q / jnp.maximum(jnp.sqrt(jnp.sum(q * q, axis=1, keepdims=True)), 1e-12)
    qb = qn.astype(jnp.bfloat16)
    s = jnp.dot(qb, kb_ref[...].T, preferred_element_type=jnp.float32)
    col_ids = jax.lax.broadcasted_iota(jnp.int32, (TILE, MP), 1)
    s = jnp.where(col_ids < M, s, -1e30)
    rowmax = jnp.max(s, axis=1, keepdims=True)
    gi = jnp.min(jnp.where(s == rowmax, col_ids, MP), axis=1, keepdims=True)
    colmax_ref[...] = jnp.maximum(colmax_ref[...], jnp.max(s, axis=0, keepdims=True))

    onehot = (col_ids == gi).astype(jnp.bfloat16)
    qw = (qn * jnp.exp(rowmax)).astype(jnp.bfloat16)
    acc_ref[...] += jax.lax.dot_general(
        qw, onehot, (((0,), (0,)), ((), ())), preferred_element_type=jnp.float32
    )

    @pl.when(t == T - 1)
    def _finish():
        upd = 1e-05 * jnp.exp(-colmax_ref[...]) * acc_ref[...] + kt_ref[...]
        norm = jnp.sqrt(jnp.sum(upd * upd, axis=0, keepdims=True))
        out_ref[...] = upd / jnp.maximum(norm, 1e-12)


@jax.jit
def kernel(query, keys):
    q2 = query.reshape(N_TOK, D)
    kp = jnp.pad(keys, ((0, MP - M), (0, 0)))
    out_t = pl.pallas_call(
        _body,
        grid=(T,),
        in_specs=[
            pl.BlockSpec((TILE, D), lambda t: (t, 0)),
            pl.BlockSpec((MP, D), lambda t: (0, 0)),
            pl.BlockSpec((D, MP), lambda t: (0, 0)),
        ],
        out_specs=pl.BlockSpec((D, MP), lambda t: (0, 0)),
        out_shape=jax.ShapeDtypeStruct((D, MP), jnp.float32),
        scratch_shapes=[
            pltpu.VMEM((1, MP), jnp.float32),
            pltpu.VMEM((D, MP), jnp.float32),
        ],
    )(q2, kp.astype(jnp.bfloat16), kp.T)
    return out_t.T[:M]


# speedup vs baseline: 18.5693x; 1.5113x over previous
import jax
import jax.numpy as jnp
from jax.experimental import pallas as pl
from jax.experimental.pallas import tpu as pltpu

N_TOK = 16 * 2048
D = 128
M = 1000
MP = 1024
TILE = 4096
T = N_TOK // TILE


def _body(q_ref, kb_ref, kt_ref, out_ref, colmax_ref, acc_ref):
    t = pl.program_id(0)

    @pl.when(t == 0)
    def _init():
        colmax_ref[...] = jnp.full((1, MP), -1e30, jnp.float32)
        acc_ref[...] = jnp.zeros((D, MP), jnp.float32)

    q = q_ref[...]
    qn = q / jnp.maximum(jnp.sqrt(jnp.sum(q * q, axis=1, keepdims=True)), 1e-12)
    qb = qn.astype(jnp.bfloat16)
    s = jnp.dot(qb, kb_ref[...].T, preferred_element_type=jnp.float32)
    col_ids = jax.lax.broadcasted_iota(jnp.int32, (TILE, MP), 1)
    s = jnp.where(col_ids < M, s, -1e30)
    rowmax = jnp.max(s, axis=1, keepdims=True)
    colmax_ref[...] = jnp.maximum(colmax_ref[...], jnp.max(s, axis=0, keepdims=True))

    onehot = (s == rowmax).astype(jnp.bfloat16)
    qw = (qn * jnp.exp(rowmax)).astype(jnp.bfloat16)
    acc_ref[...] += jax.lax.dot_general(
        qw, onehot, (((0,), (0,)), ((), ())), preferred_element_type=jnp.float32
    )

    @pl.when(t == T - 1)
    def _finish():
        upd = 1e-05 * jnp.exp(-colmax_ref[...]) * acc_ref[...] + kt_ref[...]
        norm = jnp.sqrt(jnp.sum(upd * upd, axis=0, keepdims=True))
        out_ref[...] = upd / jnp.maximum(norm, 1e-12)


@jax.jit
def kernel(query, keys):
    q2 = query.reshape(N_TOK, D)
    kp = jnp.pad(keys, ((0, MP - M), (0, 0)))
    out_t = pl.pallas_call(
        _body,
        grid=(T,),
        in_specs=[
            pl.BlockSpec((TILE, D), lambda t: (t, 0)),
            pl.BlockSpec((MP, D), lambda t: (0, 0)),
            pl.BlockSpec((D, MP), lambda t: (0, 0)),
        ],
        out_specs=pl.BlockSpec((D, MP), lambda t: (0, 0)),
        out_shape=jax.ShapeDtypeStruct((D, MP), jnp.float32),
        scratch_shapes=[
            pltpu.VMEM((1, MP), jnp.float32),
            pltpu.VMEM((D, MP), jnp.float32),
        ],
    )(q2, kp.astype(jnp.bfloat16), kp.T)
    return out_t.T[:M]


# speedup vs baseline: 19.2640x; 1.0374x over previous
import jax
import jax.numpy as jnp
from jax.experimental import pallas as pl
from jax.experimental.pallas import tpu as pltpu

N_TOK = 16 * 2048
D = 128
M = 1000
MP = 1024
TILE = 4096
T = N_TOK // TILE


def _body(q_ref, kb_ref, kt_ref, out_ref, colmax_ref, acc_ref):
    t = pl.program_id(0)

    @pl.when(t == 0)
    def _init():
        colmax_ref[...] = jnp.full((1, MP), -1e30, jnp.float32)
        acc_ref[...] = jnp.zeros((D, MP), jnp.float32)

    q = q_ref[...]
    ones = jnp.ones((D, D), jnp.bfloat16)
    ss = jnp.dot((q * q).astype(jnp.bfloat16), ones, preferred_element_type=jnp.float32)
    inv = jax.lax.rsqrt(jnp.maximum(ss, 1e-24))
    qn = q * inv
    qb = qn.astype(jnp.bfloat16)
    s = jnp.dot(qb, kb_ref[...].T, preferred_element_type=jnp.float32)
    col_ids = jax.lax.broadcasted_iota(jnp.int32, (TILE, MP), 1)
    sb = jnp.where(col_ids < M, s.astype(jnp.bfloat16), jnp.bfloat16(-1e30))
    rowmax = jnp.max(sb, axis=1, keepdims=True)
    colmax_ref[...] = jnp.maximum(
        colmax_ref[...], jnp.max(sb, axis=0, keepdims=True).astype(jnp.float32)
    )

    onehot = jnp.where(sb == rowmax, jnp.bfloat16(1), jnp.bfloat16(0))
    qw = (qn * jnp.exp(rowmax.astype(jnp.float32))).astype(jnp.bfloat16)
    acc_ref[...] += jax.lax.dot_general(
        qw, onehot, (((0,), (0,)), ((), ())), preferred_element_type=jnp.float32
    )

    @pl.when(t == T - 1)
    def _finish():
        upd = 1e-05 * jnp.exp(-colmax_ref[...]) * acc_ref[...] + kt_ref[...]
        norm = jnp.sqrt(jnp.sum(upd * upd, axis=0, keepdims=True))
        out_ref[...] = upd / jnp.maximum(norm, 1e-12)


@jax.jit
def kernel(query, keys):
    q2 = query.reshape(N_TOK, D)
    kp = jnp.pad(keys, ((0, MP - M), (0, 0)))
    out_t = pl.pallas_call(
        _body,
        grid=(T,),
        in_specs=[
            pl.BlockSpec((TILE, D), lambda t: (t, 0)),
            pl.BlockSpec((MP, D), lambda t: (0, 0)),
            pl.BlockSpec((D, MP), lambda t: (0, 0)),
        ],
        out_specs=pl.BlockSpec((D, MP), lambda t: (0, 0)),
        out_shape=jax.ShapeDtypeStruct((D, MP), jnp.float32),
        scratch_shapes=[
            pltpu.VMEM((1, MP), jnp.float32),
            pltpu.VMEM((D, MP), jnp.float32),
        ],
    )(q2, kp.astype(jnp.bfloat16), kp.T)
    return out_t.T[:M]


# speedup vs baseline: 20.7800x; 1.0787x over previous
import jax
import jax.numpy as jnp
from jax.experimental import pallas as pl
from jax.experimental.pallas import tpu as pltpu

N_TOK = 16 * 2048
D = 128
M = 1000
MP = 1024
TILE = 4096
T = N_TOK // TILE


def _body(q_ref, k_ref, out_ref, kb_ref, colmax_ref, acc_ref):
    t = pl.program_id(0)

    @pl.when(t == 0)
    def _init():
        colmax_ref[...] = jnp.full((1, MP), -1e30, jnp.float32)
        acc_ref[...] = jnp.zeros((D, MP), jnp.float32)
        kb_ref[...] = k_ref[...].astype(jnp.bfloat16)

    q = q_ref[...]
    ones = jnp.ones((D, D), jnp.bfloat16)
    ss = jnp.dot((q * q).astype(jnp.bfloat16), ones, preferred_element_type=jnp.float32)
    qn = q * jax.lax.rsqrt(jnp.maximum(ss, 1e-24))
    qb = qn.astype(jnp.bfloat16)
    s = jnp.dot(qb, kb_ref[...].T, preferred_element_type=jnp.float32)
    col_ids = jax.lax.broadcasted_iota(jnp.int32, (TILE, MP), 1)
    sb = jnp.where(col_ids < M, s.astype(jnp.bfloat16), jnp.bfloat16(-1e30))
    rowmax = jnp.max(sb, axis=1, keepdims=True)
    colmax_ref[...] = jnp.maximum(
        colmax_ref[...], jnp.max(sb, axis=0, keepdims=True).astype(jnp.float32)
    )

    onehot = jnp.where(sb == rowmax, jnp.bfloat16(1), jnp.bfloat16(0))
    qw = (qn * jnp.exp(rowmax.astype(jnp.float32))).astype(jnp.bfloat16)
    acc_ref[...] += jax.lax.dot_general(
        qw, onehot, (((0,), (0,)), ((), ())), preferred_element_type=jnp.float32
    )

    @pl.when(t == T - 1)
    def _finish():
        ut = 1e-05 * jnp.exp(-colmax_ref[...]) * acc_ref[...]
        upd = jnp.transpose(ut) + k_ref[...]
        nrm = jnp.sum(upd * upd, axis=1, keepdims=True)
        out_ref[...] = (upd * jax.lax.rsqrt(jnp.maximum(nrm, 1e-24)))[:M]


@jax.jit
def kernel(query, keys):
    q2 = query.reshape(N_TOK, D)
    kp = jnp.pad(keys, ((0, MP - M), (0, 0)))
    return pl.pallas_call(
        _body,
        grid=(T,),
        in_specs=[
            pl.BlockSpec((TILE, D), lambda t: (t, 0)),
            pl.BlockSpec((MP, D), lambda t: (0, 0)),
        ],
        out_specs=pl.BlockSpec((M, D), lambda t: (0, 0)),
        out_shape=jax.ShapeDtypeStruct((M, D), jnp.float32),
        scratch_shapes=[
            pltpu.VMEM((MP, D), jnp.bfloat16),
            pltpu.VMEM((1, MP), jnp.float32),
            pltpu.VMEM((D, MP), jnp.float32),
        ],
    )(q2, kp)
